# Initial kernel scaffold; baseline (speedup 1.0000x reference)
import jax
import jax.numpy as jnp
from jax.experimental import pallas as pl


def kernel(x, edge_index, W, att_src, att_dst, b_conv, Wa, ba, W1, b1, W2, b2):
    raise NotImplementedError("write your pallas kernel here")


# speedup vs baseline: 1.0592x; 1.0592x over previous
import functools

import jax
import jax.numpy as jnp
from jax import lax
from jax.experimental import pallas as pl
from jax.experimental.pallas import tpu as pltpu

_H = 2
_C = 256


def _tc1_body(x_ref, w_ref, as_ref, ad_ref, h0_ref, h1_ref, asrc_ref, adst_ref):
    hb = jnp.dot(x_ref[...], w_ref[...], preferred_element_type=jnp.float32)
    h0_ref[...] = hb[:, :_C]
    h1_ref[...] = hb[:, _C:]
    asrc_ref[...] = jnp.dot(hb, as_ref[...], preferred_element_type=jnp.float32)
    adst_ref[...] = jnp.dot(hb, ad_ref[...], preferred_element_type=jnp.float32)


def _tc1(x, W, As, Ad, blk=1000):
    n, d = x.shape
    grid = n // blk
    return pl.pallas_call(
        _tc1_body,
        grid=(grid,),
        in_specs=[
            pl.BlockSpec((blk, d), lambda i: (i, 0)),
            pl.BlockSpec((d, _H * _C), lambda i: (0, 0)),
            pl.BlockSpec((_H * _C, _H), lambda i: (0, 0)),
            pl.BlockSpec((_H * _C, _H), lambda i: (0, 0)),
        ],
        out_specs=[
            pl.BlockSpec((blk, _C), lambda i: (i, 0)),
            pl.BlockSpec((blk, _C), lambda i: (i, 0)),
            pl.BlockSpec((blk, _H), lambda i: (i, 0)),
            pl.BlockSpec((blk, _H), lambda i: (i, 0)),
        ],
        out_shape=[
            jax.ShapeDtypeStruct((n, _C), jnp.float32),
            jax.ShapeDtypeStruct((n, _C), jnp.float32),
            jax.ShapeDtypeStruct((n, _H), jnp.float32),
            jax.ShapeDtypeStruct((n, _H), jnp.float32),
        ],
    )(x, W, As, Ad)


def _mlp_body(o0_ref, o1_ref, bc_ref, wa_ref, ba_ref, w1_ref, b1_ref,
              w2_ref, b2_ref, a_ref, bt_ref):
    blk = o0_ref.shape[0]
    g = jnp.concatenate([o0_ref[...], o1_ref[...]], axis=1) + bc_ref[...]
    g = jnp.maximum(g, 0.0)
    t = jnp.dot(g, wa_ref[...], preferred_element_type=jnp.float32) + ba_ref[...]
    t = jnp.maximum(t, 0.0)
    t = jnp.dot(t, w1_ref[...], preferred_element_type=jnp.float32) + b1_ref[...]
    t = jnp.maximum(t, 0.0)
    z8 = jnp.dot(t, w2_ref[...], preferred_element_type=jnp.float32) + b2_ref[...]
    sq = jnp.sum(z8 * z8, axis=1, keepdims=True)
    col = lax.broadcasted_iota(jnp.int32, (blk, 8), 1)
    a_ref[...] = jnp.where(col < 3, z8,
                           jnp.where(col == 3, sq,
                                     jnp.where(col == 4, 1.0, 0.0)))
    bt_ref[...] = jnp.where(col < 3, -2.0 * z8,
                            jnp.where(col == 3, 1.0,
                                      jnp.where(col == 4, sq, 0.0)))


def _mlp(o0, o1, b_conv, Wa, ba, W1, b1, W2p, b2p, blk=1000):
    n = o0.shape[0]
    grid = n // blk
    return pl.pallas_call(
        _mlp_body,
        grid=(grid,),
        in_specs=[
            pl.BlockSpec((blk, _C), lambda i: (i, 0)),
            pl.BlockSpec((blk, _C), lambda i: (i, 0)),
            pl.BlockSpec((1, _H * _C), lambda i: (0, 0)),
            pl.BlockSpec((_H * _C, 128), lambda i: (0, 0)),
            pl.BlockSpec((1, 128), lambda i: (0, 0)),
            pl.BlockSpec((128, 32), lambda i: (0, 0)),
            pl.BlockSpec((1, 32), lambda i: (0, 0)),
            pl.BlockSpec((32, 8), lambda i: (0, 0)),
            pl.BlockSpec((1, 8), lambda i: (0, 0)),
        ],
        out_specs=[
            pl.BlockSpec((blk, 8), lambda i: (i, 0)),
            pl.BlockSpec((blk, 8), lambda i: (i, 0)),
        ],
        out_shape=[
            jax.ShapeDtypeStruct((n, 8), jnp.float32),
            jax.ShapeDtypeStruct((n, 8), jnp.float32),
        ],
    )(o0, o1, b_conv, Wa, ba, W1, b1, W2p, b2p)


def _dist_body(a_ref, bt_ref, out_ref):
    d2 = lax.dot_general(a_ref[...], bt_ref[...], (((1,), (1,)), ((), ())),
                         preferred_element_type=jnp.float32)
    out_ref[...] = jnp.where(d2 > 0.0, jnp.sqrt(jnp.where(d2 > 0.0, d2, 1.0)), 0.0)


def _dist(A, Bt, blk=400):
    n = A.shape[0]
    grid = n // blk
    return pl.pallas_call(
        _dist_body,
        grid=(grid,),
        in_specs=[
            pl.BlockSpec((blk, 8), lambda i: (i, 0)),
            pl.BlockSpec((n, 8), lambda i: (0, 0)),
        ],
        out_specs=pl.BlockSpec((blk, n), lambda i: (i, 0)),
        out_shape=jax.ShapeDtypeStruct((n, n), jnp.float32),
    )(A, Bt)


def _edge_stage(h0, h1, a_src, a_dst, src, dst, n):
    e = a_src[src] + a_dst[dst]
    e = jnp.where(e >= 0, e, 0.2 * e)
    ex = jnp.exp(e)
    denom = jax.ops.segment_sum(ex, dst, num_segments=n)
    alpha = ex / (denom[dst] + 1e-16)
    h = jnp.concatenate([h0, h1], axis=1).reshape(n, _H, _C)
    msg = h[src] * alpha[..., None]
    out = jax.ops.segment_sum(msg, dst, num_segments=n)
    return out[:, 0, :], out[:, 1, :]


def kernel(x, edge_index, W, att_src, att_dst, b_conv, Wa, ba, W1, b1, W2, b2):
    n, d = x.shape
    hc = _H * _C

    As = jnp.zeros((hc, _H), jnp.float32)
    As = As.at[:_C, 0].set(att_src[0]).at[_C:, 1].set(att_src[1])
    Ad = jnp.zeros((hc, _H), jnp.float32)
    Ad = Ad.at[:_C, 0].set(att_dst[0]).at[_C:, 1].set(att_dst[1])

    h0, h1, a_src, a_dst = _tc1(x, W, As, Ad)

    ar = jnp.arange(n, dtype=edge_index.dtype)
    src = jnp.concatenate([edge_index[0], ar])
    dst = jnp.concatenate([edge_index[1], ar])

    o0, o1 = _edge_stage(h0, h1, a_src, a_dst, src, dst, n)

    W2p = jnp.zeros((32, 8), jnp.float32).at[:, :3].set(W2)
    b2p = jnp.zeros((8,), jnp.float32).at[:3].set(b2)
    A, Bt = _mlp(o0, o1, b_conv.reshape(1, hc), Wa, ba.reshape(1, 128),
                 W1, b1.reshape(1, 32), W2p, b2p.reshape(1, 8))
    return _dist(A, Bt)


# speedup vs baseline: 7.8484x; 7.4097x over previous
import functools

import jax
import jax.numpy as jnp
from jax import lax
from jax.experimental import pallas as pl
from jax.experimental.pallas import tpu as pltpu
from jax.experimental.pallas import tpu_sc as plsc

_H = 2
_C = 256
_NPAD = 10240
_HALFN = 5120
_EPAD = 172032


def _tc1_body(x_ref, w_ref, as_ref, ad_ref, h_ref, asrc_ref, adst_ref):
    hb = jnp.dot(x_ref[...], w_ref[...], preferred_element_type=jnp.float32)
    h_ref[0] = hb[:, :_C]
    h_ref[1] = hb[:, _C:]
    asrc_ref[...] = jnp.dot(hb, as_ref[...], preferred_element_type=jnp.float32)
    adst_ref[...] = jnp.dot(hb, ad_ref[...], preferred_element_type=jnp.float32)


def _tc1(x, W, As, Ad, blk=1000):
    n, d = x.shape
    grid = n // blk
    return pl.pallas_call(
        _tc1_body,
        grid=(grid,),
        in_specs=[
            pl.BlockSpec((blk, d), lambda i: (i, 0)),
            pl.BlockSpec((d, _H * _C), lambda i: (0, 0)),
            pl.BlockSpec((_H * _C, _H), lambda i: (0, 0)),
            pl.BlockSpec((_H * _C, _H), lambda i: (0, 0)),
        ],
        out_specs=[
            pl.BlockSpec((_H, blk, _C), lambda i: (0, i, 0)),
            pl.BlockSpec((blk, _H), lambda i: (i, 0)),
            pl.BlockSpec((blk, _H), lambda i: (i, 0)),
        ],
        out_shape=[
            jax.ShapeDtypeStruct((_H, n, _C), jnp.float32),
            jax.ShapeDtypeStruct((n, _H), jnp.float32),
            jax.ShapeDtypeStruct((n, _H), jnp.float32),
        ],
    )(x, W, As, Ad)


def _sc_softmax_denom(a_src, a_dst, srcp, dstp, e_valid):
    n = a_src.shape[0] // _H
    ach = _EPAD // 32

    @functools.partial(
        pl.kernel,
        out_type=[
            jax.ShapeDtypeStruct((_H * _EPAD,), jnp.float32),
            jax.ShapeDtypeStruct((2 * _H * _NPAD,), jnp.float32),
        ],
        mesh=plsc.VectorSubcoreMesh(core_axis_name="c", subcore_axis_name="s"),
        compiler_params=pltpu.CompilerParams(needs_layout_passes=False),
        scratch_types=[
            pltpu.VMEM((n * _H,), jnp.float32),
            pltpu.VMEM((n * _H,), jnp.float32),
            pltpu.VMEM((_NPAD,), jnp.float32),
            pltpu.VMEM((_NPAD,), jnp.float32),
            pltpu.VMEM((ach,), jnp.int32),
            pltpu.VMEM((ach,), jnp.int32),
            pltpu.VMEM((ach,), jnp.float32),
            pltpu.VMEM((ach,), jnp.float32),
            pltpu.VMEM((_NPAD // 16,), jnp.float32),
            pltpu.VMEM((_NPAD // 16,), jnp.float32),
            pltpu.VMEM_SHARED((16, _H, _NPAD), jnp.float32),
        ],
    )
    def k(asrc_hbm, adst_hbm, src_hbm, dst_hbm, ex_hbm, dnp_hbm,
          asv, adv, dn0, dn1, srcv, dstv, ex0v, ex1v, accv, tmpv, stage):
        c = lax.axis_index("c")
        s = lax.axis_index("s")
        w = s * 2 + c
        base = w * ach
        pltpu.sync_copy(asrc_hbm, asv)
        pltpu.sync_copy(adst_hbm, adv)
        pltpu.sync_copy(src_hbm.at[pl.ds(base, ach)], srcv)
        pltpu.sync_copy(dst_hbm.at[pl.ds(base, ach)], dstv)

        zf = jnp.zeros((16,), jnp.float32)

        def zb(i, _):
            dn0[pl.ds(i * 16, 16)] = zf
            dn1[pl.ds(i * 16, 16)] = zf
            return 0
        lax.fori_loop(0, _NPAD // 16, zb, 0)

        iot = lax.iota(jnp.int32, 16)

        def body(i, _):
            off = i * 16
            sv = srcv[pl.ds(off, 16)]
            dv = dstv[pl.ds(off, 16)]
            sv2 = sv + sv
            dv2 = dv + dv
            as0 = plsc.load_gather(asv, [sv2])
            as1 = plsc.load_gather(asv, [sv2 + 1])
            ad0 = plsc.load_gather(adv, [dv2])
            ad1 = plsc.load_gather(adv, [dv2 + 1])
            valid = (base + off + iot) < e_valid
            e0 = as0 + ad0
            e0 = jnp.maximum(e0, 0.2 * e0)
            x0 = jnp.where(valid, jnp.exp(e0), 0.0)
            e1 = as1 + ad1
            e1 = jnp.maximum(e1, 0.2 * e1)
            x1 = jnp.where(valid, jnp.exp(e1), 0.0)
            ex0v[pl.ds(off, 16)] = x0
            ex1v[pl.ds(off, 16)] = x1
            plsc.addupdate_scatter(dn0, [dv], x0)
            plsc.addupdate_scatter(dn1, [dv], x1)
            return 0
        lax.fori_loop(0, ach // 16, body, 0)

        pltpu.sync_copy(ex0v, ex_hbm.at[pl.ds(base, ach)])
        pltpu.sync_copy(ex1v, ex_hbm.at[pl.ds(_EPAD + base, ach)])

        pltpu.sync_copy(dn0, stage.at[s, 0])
        pltpu.sync_copy(dn1, stage.at[s, 1])
        plsc.subcore_barrier()
        nslc = _NPAD // 16
        lo = s * nslc
        for h in range(_H):
            def zacc(i, _):
                accv[pl.ds(i * 16, 16)] = zf
                return 0
            lax.fori_loop(0, nslc // 16, zacc, 0)

            def red(k2, _):
                pltpu.sync_copy(stage.at[k2, h, pl.ds(lo, nslc)], tmpv)

                def addv(v, _):
                    sl = pl.ds(v * 16, 16)
                    accv[sl] = accv[sl] + tmpv[sl]
                    return 0
                lax.fori_loop(0, nslc // 16, addv, 0)
                return 0
            lax.fori_loop(0, 16, red, 0)
            pltpu.sync_copy(accv, dnp_hbm.at[pl.ds((c + c + h) * _NPAD + lo, nslc)])

    return k(a_src, a_dst, srcp, dstp)


def _sc_aggregate(h_stk, srcp, dstp, ex, dnp, n):
    bch = _EPAD // 16
    sube = 1344
    nsub_a = bch // sube
    nsub_s = _EPAD // sube
    grp = 64
    cap = 2048
    win = _NPAD // 32
    arows = win + 8
    dchunk = 2000

    @functools.partial(
        pl.kernel,
        out_type=[
            jax.ShapeDtypeStruct((_H * _NPAD * _C,), jnp.float32),
            jax.ShapeDtypeStruct((_H * _EPAD,), jnp.float32),
        ],
        mesh=plsc.VectorSubcoreMesh(core_axis_name="c", subcore_axis_name="s"),
        compiler_params=pltpu.CompilerParams(needs_layout_passes=False),
        scratch_types=[
            pltpu.VMEM((n,), jnp.float32),
            pltpu.VMEM((dchunk,), jnp.float32),
            pltpu.VMEM((sube,), jnp.int32),
            pltpu.VMEM((sube,), jnp.int32),
            pltpu.VMEM((sube,), jnp.float32),
            pltpu.VMEM((arows * _C,), jnp.float32),
            pltpu.VMEM((cap // grp, grp), jnp.int32),
            pltpu.VMEM((cap,), jnp.int32),
            pltpu.VMEM((cap,), jnp.float32),
            pltpu.VMEM((grp, _C), jnp.float32),
            pltpu.SemaphoreType.DMA,
        ],
    )
    def k(h_hbm, src_hbm, dst_hbm, ex_hbm, dnp_hbm, out_hbm, al_hbm,
          dnv, dtv, ssub, dsub, aasub, acc, psrc, prel, pal, rowb, sem):
        c = lax.axis_index("c")
        s = lax.axis_index("s")
        zf16 = jnp.zeros((16,), jnp.float32)
        zi16 = jnp.zeros((16,), jnp.int32)
        iot = lax.iota(jnp.int32, 16)

        pltpu.sync_copy(dnp_hbm.at[pl.ds(c * _NPAD, n)], dnv)
        for q in range(n // dchunk):
            pltpu.sync_copy(dnp_hbm.at[pl.ds((c + 2) * _NPAD + q * dchunk, dchunk)], dtv)

            def addv(v, _, q=q):
                sl = pl.ds(q * dchunk + v * 16, 16)
                dnv[sl] = dnv[sl] + dtv[pl.ds(v * 16, 16)]
                return 0
            lax.fori_loop(0, dchunk // 16, addv, 0)

        base = s * bch

        def alsub(q2, _):
            off0 = base + q2 * sube
            pltpu.sync_copy(ex_hbm.at[pl.ds(c * _EPAD + off0, sube)], aasub)
            pltpu.sync_copy(dst_hbm.at[pl.ds(off0, sube)], dsub)

            def albody(i, _):
                off = i * 16
                dv = dsub[pl.ds(off, 16)]
                dn = plsc.load_gather(dnv, [dv])
                sl = pl.ds(off, 16)
                aasub[sl] = aasub[sl] / (dn + 1e-16)
                return 0
            lax.fori_loop(0, sube // 16, albody, 0)
            pltpu.sync_copy(aasub, al_hbm.at[pl.ds(c * _EPAD + off0, sube)])
            return 0
        lax.fori_loop(0, nsub_a, alsub, 0)
        plsc.subcore_barrier()

        for p in range(2):
            wlo = p * (_NPAD // 2) + s * win

            def zacc(i, _):
                acc[pl.ds(i * 16, 16)] = zf16
                return 0
            lax.fori_loop(0, (arows * _C) // 16, zacc, 0)

            def dogroup(j, cend):
                jr = lax.bitwise_and(j, (cap // grp) - 1)
                pltpu.async_copy(
                    h_hbm.at[c].at[psrc.at[jr]], rowb, sem).wait()

                def srow(r, _):
                    pp = jr * grp + r
                    a = plsc.load_gather(pal, [zi16 + pp])
                    relv = plsc.load_gather(prel, [zi16 + pp])

                    def svec(v, _):
                        sl = pl.ds(v * 16, 16)
                        idxv = relv * _C + (v * 16 + iot)
                        plsc.addupdate_scatter(acc, [idxv], rowb[r, sl] * a)
                        return 0
                    lax.fori_loop(0, _C // 16, svec, 0)
                    return 0
                lax.fori_loop(0, grp, srow, 0)
                return cend

            def scansub(q2, carry):
                cnt, gdone = carry
                off0 = q2 * sube
                pltpu.sync_copy(src_hbm.at[pl.ds(off0, sube)], ssub)
                pltpu.sync_copy(dst_hbm.at[pl.ds(off0, sube)], dsub)
                pltpu.sync_copy(al_hbm.at[pl.ds(c * _EPAD + off0, sube)], aasub)

                def pack(i, cnt, wlo=wlo):
                    off = i * 16
                    dv = dsub[pl.ds(off, 16)]
                    m = (dv >= wlo) & (dv < wlo + win)
                    mi = m.astype(jnp.int32)
                    pos = jnp.maximum(cnt + plsc.cumsum(mi) - 1, 0)
                    posw = lax.bitwise_and(pos, cap - 1)
                    rowi = lax.shift_right_logical(posw, 6)
                    coli = lax.bitwise_and(posw, grp - 1)
                    plsc.store_scatter(psrc, [rowi, coli], ssub[pl.ds(off, 16)], mask=m)
                    plsc.store_scatter(prel, [posw], dv - wlo, mask=m)
                    plsc.store_scatter(pal, [posw], aasub[pl.ds(off, 16)], mask=m)
                    return cnt + jnp.sum(mi)
                cnt = lax.fori_loop(0, sube // 16, pack, cnt)
                gdone = lax.fori_loop(gdone, lax.shift_right_logical(cnt, 6),
                                      dogroup, gdone)
                return cnt, lax.shift_right_logical(cnt, 6)

            cnt, gdone = lax.fori_loop(0, nsub_s, scansub,
                                       (jnp.int32(0), jnp.int32(0)))

            for kk in range(grp // 16):
                posd = lax.bitwise_and(cnt + kk * 16 + iot, cap - 1)
                plsc.store_scatter(psrc,
                                   [lax.shift_right_logical(posd, 6),
                                    lax.bitwise_and(posd, grp - 1)], zi16)
                plsc.store_scatter(prel, [posd], zi16 + win)
                plsc.store_scatter(pal, [posd], zf16)
            lax.fori_loop(gdone, lax.shift_right_logical(cnt + grp - 1, 6),
                          dogroup, gdone)

            pltpu.sync_copy(acc.at[pl.ds(0, win * _C)],
                            out_hbm.at[pl.ds(c * (_NPAD * _C) + wlo * _C, win * _C)])

    return k(h_stk, srcp, dstp, ex, dnp)


def _mlp_body(o0_ref, o1_ref, bc_ref, wa_ref, ba_ref, w1_ref, b1_ref,
              w2_ref, b2_ref, a_ref, bt_ref):
    blk = o0_ref.shape[1]
    g = jnp.concatenate([o0_ref[0], o1_ref[0]], axis=1) + bc_ref[...]
    g = jnp.maximum(g, 0.0)
    t = jnp.dot(g, wa_ref[...], preferred_element_type=jnp.float32) + ba_ref[...]
    t = jnp.maximum(t, 0.0)
    t = jnp.dot(t, w1_ref[...], preferred_element_type=jnp.float32) + b1_ref[...]
    t = jnp.maximum(t, 0.0)
    z8 = jnp.dot(t, w2_ref[...], preferred_element_type=jnp.float32) + b2_ref[...]
    sq = jnp.sum(z8 * z8, axis=1, keepdims=True)
    col = lax.broadcasted_iota(jnp.int32, (blk, 8), 1)
    a_ref[...] = jnp.where(col < 3, z8,
                           jnp.where(col == 3, sq,
                                     jnp.where(col == 4, 1.0, 0.0)))
    bt_ref[...] = jnp.where(col < 3, -2.0 * z8,
                            jnp.where(col == 3, 1.0,
                                      jnp.where(col == 4, sq, 0.0)))


def _mlp(out_conv, b_conv, Wa, ba, W1, b1, W2p, b2p, n, blk=1000):
    grid = n // blk
    return pl.pallas_call(
        _mlp_body,
        grid=(grid,),
        in_specs=[
            pl.BlockSpec((1, blk, _C), lambda i: (0, i, 0)),
            pl.BlockSpec((1, blk, _C), lambda i: (1, i, 0)),
            pl.BlockSpec((1, _H * _C), lambda i: (0, 0)),
            pl.BlockSpec((_H * _C, 128), lambda i: (0, 0)),
            pl.BlockSpec((1, 128), lambda i: (0, 0)),
            pl.BlockSpec((128, 32), lambda i: (0, 0)),
            pl.BlockSpec((1, 32), lambda i: (0, 0)),
            pl.BlockSpec((32, 8), lambda i: (0, 0)),
            pl.BlockSpec((1, 8), lambda i: (0, 0)),
        ],
        out_specs=[
            pl.BlockSpec((blk, 8), lambda i: (i, 0)),
            pl.BlockSpec((blk, 8), lambda i: (i, 0)),
        ],
        out_shape=[
            jax.ShapeDtypeStruct((n, 8), jnp.float32),
            jax.ShapeDtypeStruct((n, 8), jnp.float32),
        ],
    )(out_conv, out_conv, b_conv, Wa, ba, W1, b1, W2p, b2p)


def _dist_body(a_ref, bt_ref, out_ref):
    d2 = lax.dot_general(a_ref[...], bt_ref[...], (((1,), (1,)), ((), ())),
                         preferred_element_type=jnp.float32)
    out_ref[...] = jnp.where(d2 > 0.0, jnp.sqrt(jnp.where(d2 > 0.0, d2, 1.0)), 0.0)


def _dist(A, Bt, blk=400):
    n = A.shape[0]
    grid = n // blk
    return pl.pallas_call(
        _dist_body,
        grid=(grid,),
        in_specs=[
            pl.BlockSpec((blk, 8), lambda i: (i, 0)),
            pl.BlockSpec((n, 8), lambda i: (0, 0)),
        ],
        out_specs=pl.BlockSpec((blk, n), lambda i: (i, 0)),
        out_shape=jax.ShapeDtypeStruct((n, n), jnp.float32),
    )(A, Bt)


def kernel(x, edge_index, W, att_src, att_dst, b_conv, Wa, ba, W1, b1, W2, b2):
    n, d = x.shape
    hc = _H * _C
    e = edge_index.shape[1]
    e_valid = e + n

    As = jnp.zeros((hc, _H), jnp.float32)
    As = As.at[:_C, 0].set(att_src[0]).at[_C:, 1].set(att_src[1])
    Ad = jnp.zeros((hc, _H), jnp.float32)
    Ad = Ad.at[:_C, 0].set(att_dst[0]).at[_C:, 1].set(att_dst[1])

    h_stk, a_src, a_dst = _tc1(x, W, As, Ad)

    ar = jnp.arange(n, dtype=edge_index.dtype)
    pad = jnp.zeros((_EPAD - e_valid,), edge_index.dtype)
    srcp = jnp.concatenate([edge_index[0], ar, pad])
    dstp = jnp.concatenate([edge_index[1], ar, pad])

    ex, dnp = _sc_softmax_denom(a_src.reshape(-1), a_dst.reshape(-1), srcp, dstp, e_valid)
    out_flat, _alpha = _sc_aggregate(h_stk, srcp, dstp, ex, dnp, n)
    out_conv = out_flat.reshape(_H, _NPAD, _C)

    W2p = jnp.zeros((32, 8), jnp.float32).at[:, :3].set(W2)
    b2p = jnp.zeros((8,), jnp.float32).at[:3].set(b2)
    A, Bt = _mlp(out_conv, b_conv.reshape(1, hc), Wa, ba.reshape(1, 128),
                 W1, b1.reshape(1, 32), W2p, b2p.reshape(1, 8), n)
    return _dist(A, Bt)
